# baseline (speedup 1.0000x reference)
import functools

import jax
import jax.numpy as jnp
from jax import lax
from jax.experimental import pallas as pl
from jax.experimental.pallas import tpu as pltpu

PH, PW = 2, 2


def _unpool_body(val_ref, mask_ref, out_ref, *, hb, W, C, H):
    Ho, Wo = H * PH, W * PW
    b = pl.program_id(0)
    h = pl.program_id(1)
    v = val_ref[0]
    m = mask_ref[0]
    ix = lax.broadcasted_iota(jnp.int32, (PH * hb, C, Wo), 2) // PW
    vv = jnp.take_along_axis(jnp.repeat(v, PH, axis=0), ix, axis=2)
    mm = jnp.take_along_axis(jnp.repeat(m, PH, axis=0), ix, axis=2)
    Y = lax.broadcasted_iota(jnp.int32, (PH * hb, C, Wo), 0)
    c = lax.broadcasted_iota(jnp.int32, (PH * hb, C, Wo), 1)
    X = lax.broadcasted_iota(jnp.int32, (PH * hb, C, Wo), 2)
    Yg = h * (PH * hb) + Y
    oidx = ((b * Ho + Yg) * Wo + X) * C + c
    out_ref[0] = jnp.where(mm == oidx, vv, jnp.zeros_like(vv))


def _unpool_tc(val, mask, interpret=False, hb=16):
    B, H, W, C = val.shape
    Ho, Wo = H * PH, W * PW
    vt = val.transpose(0, 1, 3, 2)
    mt = mask.astype(jnp.int32).transpose(0, 1, 3, 2)
    assert H % hb == 0
    body = functools.partial(_unpool_body, hb=hb, W=W, C=C, H=H)
    out_t = pl.pallas_call(
        body,
        grid=(B, H // hb),
        in_specs=[
            pl.BlockSpec((1, hb, C, W), lambda b, h: (b, h, 0, 0)),
            pl.BlockSpec((1, hb, C, W), lambda b, h: (b, h, 0, 0)),
        ],
        out_specs=pl.BlockSpec((1, PH * hb, C, Wo), lambda b, h: (b, h, 0, 0)),
        out_shape=jax.ShapeDtypeStruct((B, Ho, C, Wo), val.dtype),
        compiler_params=pltpu.CompilerParams(
            dimension_semantics=("parallel", "parallel")),
        interpret=interpret,
    )(vt, mt)
    return out_t.transpose(0, 1, 3, 2)


def kernel(val, mask):
    return _unpool_tc(val, mask, hb=56)
